# baseline (speedup 1.0000x reference)
import functools

import jax
import jax.numpy as jnp
from jax import lax
from jax.experimental import pallas as pl
from jax.experimental.pallas import tpu as pltpu
from jax.experimental.pallas import tpu_sc as plsc

VOCAB = 128
EMBED = 32
BATCH = 16384
TIME = 200

TG, TI = 25, 8
BT, BI = 128, 128
FG, FI = 4, 8
QB = 16
NQ = BT // QB
N_UNITS = TIME * NQ


def _fuse_table_body(table_ref, w_ref, b_ref, out_ref):
    prod = lax.dot_general(
        w_ref[...], table_ref[...],
        dimension_numbers=(((1,), (1,)), ((), ())),
        preferred_element_type=jnp.float32,
    )
    out_ref[...] = jnp.maximum(prod + b_ref[...], 0.0)


def _fused_table(table, W, b):
    return pl.pallas_call(
        _fuse_table_body,
        out_shape=jax.ShapeDtypeStruct((EMBED, VOCAB), jnp.float32),
    )(table, W, b.reshape(EMBED, 1))


@functools.cache
def _make_gather():
    info = plsc.get_sparse_core_info()
    nc, ns = info.num_cores, info.num_subcores
    nw = nc * ns
    assert N_UNITS % nw == 0
    per_w = N_UNITS // nw

    mesh = plsc.VectorSubcoreMesh(core_axis_name="c", subcore_axis_name="s")

    @functools.partial(
        pl.kernel,
        mesh=mesh,
        out_type=jax.ShapeDtypeStruct((TIME, FG, BT, FI, BI), jnp.float32),
        scratch_types=[
            pltpu.VMEM((VOCAB * EMBED,), jnp.float32),
            pltpu.VMEM((2, QB, BI), jnp.int32),
            pltpu.VMEM((FG, QB, FI, BI), jnp.float32),
            pltpu.SemaphoreType.DMA,
            pltpu.SemaphoreType.DMA((2,)),
            pltpu.SemaphoreType.DMA((FG,)),
        ],
        compiler_params=pltpu.CompilerParams(
            use_tc_tiling_on_sc=False, needs_layout_passes=False),
    )
    def gather_k(ftab_hbm, idx_hbm, out_hbm, ftab_v, idx_v, out_v,
                 sem_l, sem_i, sem_o):
        wid = lax.axis_index("s") * nc + lax.axis_index("c")
        g0 = wid * per_w

        def unit_coords(g):
            t = g // NQ
            q = g % NQ
            return t // TI, t % TI, t, q

        def idx_copy(g, buf):
            tg, ti, _, q = unit_coords(g)
            return pltpu.make_async_copy(
                idx_hbm.at[tg, pl.ds(q * QB, QB), ti], idx_v.at[buf],
                sem_i.at[buf])

        def out_copy(g, fg):
            _, _, t, q = unit_coords(g)
            return pltpu.make_async_copy(
                out_v.at[fg], out_hbm.at[t, fg, pl.ds(q * QB, QB)],
                sem_o.at[fg])

        pltpu.sync_copy(ftab_hbm, ftab_v)
        idx_copy(g0, 0).start()

        def unit(g, buf, have_prev_store, prefetch):
            idx_copy(g, buf).wait()
            if prefetch is not None:
                idx_copy(g + 1, 1 - buf).start()
            else:
                @pl.when(g + 1 < g0 + per_w)
                def _():
                    idx_copy(g + 1, 1 - buf).start()
            for fg in range(FG):
                @pl.when(have_prev_store)
                def _():
                    out_copy(g, fg).wait()

                @plsc.parallel_loop(0, QB)
                def bt_body(bl):
                    for jp in range(0, FI, 4):
                        ivs = [idx_v[buf, bl, pl.ds((jp + u) * 16, 16)]
                               for u in range(4)]
                        vals = [
                            plsc.load_gather(
                                ftab_v, [ivs[u] + (fg * FI + fi) * VOCAB])
                            for u in range(4) for fi in range(FI)
                        ]
                        for u in range(4):
                            for fi in range(FI):
                                out_v[fg, bl, fi, pl.ds((jp + u) * 16, 16)] = (
                                    vals[u * FI + fi])

                out_copy(g, fg).start()

        def body(k, carry):
            g = g0 + k * 2
            unit(g, 0, k > 0, True)
            unit(g + 1, 1, jnp.bool_(True), None)
            return carry

        lax.fori_loop(0, per_w // 2, body, 0)
        g_last = g0 + per_w - 1
        for fg in range(FG):
            out_copy(g_last, fg).wait()

    return gather_k


def kernel(x, table, W, b):
    ftab = _fused_table(table, W, b).reshape(VOCAB * EMBED)
    xl = (x.reshape(BT, BI, TG, TI).transpose(2, 0, 3, 1)
          .astype(jnp.int32))
    out5 = _make_gather()(ftab, xl)
    return out5.transpose(2, 4, 0, 1, 3).reshape(BATCH, TIME, EMBED)


# speedup vs baseline: 1.0408x; 1.0408x over previous
import functools

import jax
import jax.numpy as jnp
from jax import lax
from jax.experimental import pallas as pl
from jax.experimental.pallas import tpu as pltpu
from jax.experimental.pallas import tpu_sc as plsc

VOCAB = 128
EMBED = 32
BATCH = 16384
TIME = 200

TG, TI = 25, 8
BT, BI = 128, 128
FG, FI = 4, 8
QB = 16
NQ = BT // QB
N_UNITS = TIME * NQ


def _fuse_table_body(table_ref, w_ref, b_ref, out_ref):
    prod = lax.dot_general(
        w_ref[...], table_ref[...],
        dimension_numbers=(((1,), (1,)), ((), ())),
        preferred_element_type=jnp.float32,
    )
    out_ref[...] = jnp.maximum(prod + b_ref[...], 0.0)


def _fused_table(table, W, b):
    return pl.pallas_call(
        _fuse_table_body,
        out_shape=jax.ShapeDtypeStruct((EMBED, VOCAB), jnp.float32),
    )(table, W, b.reshape(EMBED, 1))


@functools.cache
def _make_gather():
    info = plsc.get_sparse_core_info()
    nc, ns = info.num_cores, info.num_subcores
    nw = nc * ns
    assert N_UNITS % nw == 0
    per_w = N_UNITS // nw

    mesh = plsc.VectorSubcoreMesh(core_axis_name="c", subcore_axis_name="s")

    @functools.partial(
        pl.kernel,
        mesh=mesh,
        out_type=jax.ShapeDtypeStruct((TIME, FG, BT, FI, BI), jnp.float32),
        scratch_types=[
            pltpu.VMEM((VOCAB * EMBED,), jnp.float32),
            pltpu.VMEM((2, QB, BI), jnp.int32),
            pltpu.VMEM((FG, QB, FI, BI), jnp.float32),
            pltpu.SemaphoreType.DMA,
            pltpu.SemaphoreType.DMA((2,)),
            pltpu.SemaphoreType.DMA((FG,)),
        ],
        compiler_params=pltpu.CompilerParams(
            use_tc_tiling_on_sc=False, needs_layout_passes=False),
    )
    def gather_k(ftab_hbm, idx_hbm, out_hbm, ftab_v, idx_v, out_v,
                 sem_l, sem_i, sem_o):
        wid = lax.axis_index("s") * nc + lax.axis_index("c")
        g0 = wid * per_w

        def unit_coords(g):
            t = g // NQ
            q = g % NQ
            return t // TI, t % TI, t, q

        def idx_copy(g, buf):
            tg, ti, _, q = unit_coords(g)
            return pltpu.make_async_copy(
                idx_hbm.at[tg, pl.ds(q * QB, QB), ti], idx_v.at[buf],
                sem_i.at[buf])

        def out_copy(g, fg):
            _, _, t, q = unit_coords(g)
            return pltpu.make_async_copy(
                out_v.at[fg], out_hbm.at[t, fg, pl.ds(q * QB, QB)],
                sem_o.at[fg])

        pltpu.sync_copy(ftab_hbm, ftab_v)
        idx_copy(g0, 0).start()

        def unit(g, buf, have_prev_store, prefetch):
            idx_copy(g, buf).wait()
            if prefetch is not None:
                idx_copy(g + 1, 1 - buf).start()
            else:
                @pl.when(g + 1 < g0 + per_w)
                def _():
                    idx_copy(g + 1, 1 - buf).start()
            for fg in range(FG):
                @pl.when(have_prev_store)
                def _():
                    out_copy(g, fg).wait()

                @plsc.parallel_loop(0, QB)
                def bt_body(bl):
                    for jp in range(0, FI, 2):
                        ivs = [idx_v[buf, bl, pl.ds((jp + u) * 16, 16)]
                               for u in range(2)]
                        vals = [
                            plsc.load_gather(
                                ftab_v.at[pl.ds((fg * FI + fi) * VOCAB, VOCAB)],
                                [ivs[u]])
                            for u in range(2) for fi in range(FI)
                        ]
                        for u in range(2):
                            for fi in range(FI):
                                out_v[fg, bl, fi, pl.ds((jp + u) * 16, 16)] = (
                                    vals[u * FI + fi])

                out_copy(g, fg).start()

        def body(k, carry):
            g = g0 + k * 2
            unit(g, 0, k > 0, True)
            unit(g + 1, 1, jnp.bool_(True), None)
            return carry

        lax.fori_loop(0, per_w // 2, body, 0)
        g_last = g0 + per_w - 1
        for fg in range(FG):
            out_copy(g_last, fg).wait()

    return gather_k


def kernel(x, table, W, b):
    ftab = _fused_table(table, W, b).reshape(VOCAB * EMBED)
    xl = (x.reshape(BT, BI, TG, TI).transpose(2, 0, 3, 1)
          .astype(jnp.int32))
    out5 = _make_gather()(ftab, xl)
    return out5.transpose(2, 4, 0, 1, 3).reshape(BATCH, TIME, EMBED)


# speedup vs baseline: 1.2474x; 1.1985x over previous
import functools

import jax
import jax.numpy as jnp
from jax import lax
from jax.experimental import pallas as pl
from jax.experimental.pallas import tpu as pltpu
from jax.experimental.pallas import tpu_sc as plsc

VOCAB = 128
EMBED = 32
BATCH = 16384
TIME = 200

TG, TI = 25, 8
BT, BI = 128, 128
FG, FI = 4, 8
QB = 16
NQ = BT // QB
N_UNITS = TIME * NQ


def _fuse_table_body(table_ref, w_ref, b_ref, out_ref):
    prod = lax.dot_general(
        w_ref[...], table_ref[...],
        dimension_numbers=(((1,), (1,)), ((), ())),
        preferred_element_type=jnp.float32,
    )
    out_ref[...] = jnp.maximum(prod + b_ref[...], 0.0)


def _fused_table(table, W, b):
    return pl.pallas_call(
        _fuse_table_body,
        out_shape=jax.ShapeDtypeStruct((EMBED, VOCAB), jnp.float32),
    )(table, W, b.reshape(EMBED, 1))


@functools.cache
def _make_gather():
    info = plsc.get_sparse_core_info()
    nc, ns = info.num_cores, info.num_subcores
    nw = nc * ns
    assert N_UNITS % nw == 0
    per_w = N_UNITS // nw

    mesh = plsc.VectorSubcoreMesh(core_axis_name="c", subcore_axis_name="s")

    @functools.partial(
        pl.kernel,
        mesh=mesh,
        out_type=jax.ShapeDtypeStruct((TIME, FG, BT, FI, BI), jnp.float32),
        scratch_types=[
            pltpu.VMEM((VOCAB * EMBED,), jnp.float32),
            pltpu.VMEM((2, QB, BI), jnp.int32),
            pltpu.VMEM((FG, QB, FI, BI), jnp.float32),
            pltpu.SemaphoreType.DMA,
            pltpu.SemaphoreType.DMA((2,)),
            pltpu.SemaphoreType.DMA((FG,)),
        ],
        compiler_params=pltpu.CompilerParams(
            use_tc_tiling_on_sc=False, needs_layout_passes=False),
    )
    def gather_k(ftab_hbm, idx_hbm, out_hbm, ftab_v, idx_v, out_v,
                 sem_l, sem_i, sem_o):
        wid = lax.axis_index("s") * nc + lax.axis_index("c")
        g0 = wid * per_w

        def unit_coords(g):
            t = g // NQ
            q = g % NQ
            return t // TI, t % TI, t, q

        def idx_copy(g, buf):
            tg, ti, _, q = unit_coords(g)
            return pltpu.make_async_copy(
                idx_hbm.at[tg, pl.ds(q * QB, QB), ti], idx_v.at[buf],
                sem_i.at[buf])

        def out_copy(g, fg):
            _, _, t, q = unit_coords(g)
            return pltpu.make_async_copy(
                out_v.at[fg], out_hbm.at[t, fg, pl.ds(q * QB, QB)],
                sem_o.at[fg])

        pltpu.sync_copy(ftab_hbm, ftab_v)
        idx_copy(g0, 0).start()

        def unit(g, buf, have_prev_store, prefetch):
            idx_copy(g, buf).wait()
            if prefetch is not None:
                idx_copy(g + 1, 1 - buf).start()
            else:
                @pl.when(g + 1 < g0 + per_w)
                def _():
                    idx_copy(g + 1, 1 - buf).start()
            for fg in range(FG):
                @pl.when(have_prev_store)
                def _():
                    out_copy(g, fg).wait()

                @plsc.parallel_loop(0, QB)
                def bt_body(bl):
                    ivs = [idx_v[buf, bl, pl.ds(j * 16, 16)]
                           for j in range(FI)]
                    lookahead = 8
                    n = FI * FI

                    def gath(k):
                        j, fi = divmod(k, FI)
                        return plsc.load_gather(
                            ftab_v.at[pl.ds((fg * FI + fi) * VOCAB, VOCAB)],
                            [ivs[j]])

                    def store(k, val):
                        j, fi = divmod(k, FI)
                        out_v[fg, bl, fi, pl.ds(j * 16, 16)] = val

                    pend = [gath(k) for k in range(lookahead)]
                    for k in range(lookahead, n):
                        pend.append(gath(k))
                        store(k - lookahead, pend[k - lookahead])
                    for k in range(n - lookahead, n):
                        store(k, pend[k])

                out_copy(g, fg).start()

        def body(k, carry):
            g = g0 + k * 2
            unit(g, 0, k > 0, True)
            unit(g + 1, 1, jnp.bool_(True), None)
            return carry

        lax.fori_loop(0, per_w // 2, body, 0)
        g_last = g0 + per_w - 1
        for fg in range(FG):
            out_copy(g_last, fg).wait()

    return gather_k


def kernel(x, table, W, b):
    ftab = _fused_table(table, W, b).reshape(VOCAB * EMBED)
    xl = (x.reshape(BT, BI, TG, TI).transpose(2, 0, 3, 1)
          .astype(jnp.int32))
    out5 = _make_gather()(ftab, xl)
    return out5.transpose(2, 4, 0, 1, 3).reshape(BATCH, TIME, EMBED)


# speedup vs baseline: 1.2556x; 1.0066x over previous
import functools

import jax
import jax.numpy as jnp
from jax import lax
from jax.experimental import pallas as pl
from jax.experimental.pallas import tpu as pltpu
from jax.experimental.pallas import tpu_sc as plsc

VOCAB = 128
EMBED = 32
BATCH = 16384
TIME = 200

TG, TI = 25, 8
BT, BI = 128, 128
FG, FI = 4, 8
QB = 32
NQ = BT // QB
N_UNITS = TIME * NQ


def _fuse_table_body(table_ref, w_ref, b_ref, out_ref):
    prod = lax.dot_general(
        w_ref[...], table_ref[...],
        dimension_numbers=(((1,), (1,)), ((), ())),
        preferred_element_type=jnp.float32,
    )
    out_ref[...] = jnp.maximum(prod + b_ref[...], 0.0)


def _fused_table(table, W, b):
    return pl.pallas_call(
        _fuse_table_body,
        out_shape=jax.ShapeDtypeStruct((EMBED, VOCAB), jnp.float32),
    )(table, W, b.reshape(EMBED, 1))


@functools.cache
def _make_gather():
    info = plsc.get_sparse_core_info()
    nc, ns = info.num_cores, info.num_subcores
    nw = nc * ns
    assert N_UNITS % nw == 0
    per_w = N_UNITS // nw

    mesh = plsc.VectorSubcoreMesh(core_axis_name="c", subcore_axis_name="s")

    @functools.partial(
        pl.kernel,
        mesh=mesh,
        out_type=jax.ShapeDtypeStruct((TIME, FG, BT, FI, BI), jnp.float32),
        scratch_types=[
            pltpu.VMEM((VOCAB * EMBED,), jnp.float32),
            pltpu.VMEM((2, QB, BI), jnp.int32),
            pltpu.VMEM((2, QB, FI, BI), jnp.float32),
            pltpu.SemaphoreType.DMA,
            pltpu.SemaphoreType.DMA((2,)),
            pltpu.SemaphoreType.DMA((2,)),
        ],
        compiler_params=pltpu.CompilerParams(
            use_tc_tiling_on_sc=False, needs_layout_passes=False),
    )
    def gather_k(ftab_hbm, idx_hbm, out_hbm, ftab_v, idx_v, out_v,
                 sem_l, sem_i, sem_o):
        wid = lax.axis_index("s") * nc + lax.axis_index("c")
        g0 = wid * per_w

        def unit_coords(g):
            t = g // NQ
            q = g % NQ
            return t // TI, t % TI, t, q

        def idx_copy(g, buf):
            tg, ti, _, q = unit_coords(g)
            return pltpu.make_async_copy(
                idx_hbm.at[tg, pl.ds(q * QB, QB), ti], idx_v.at[buf],
                sem_i.at[buf])

        def out_copy(g, fg):
            _, _, t, q = unit_coords(g)
            return pltpu.make_async_copy(
                out_v.at[fg % 2], out_hbm.at[t, fg, pl.ds(q * QB, QB)],
                sem_o.at[fg % 2])

        pltpu.sync_copy(ftab_hbm, ftab_v)
        idx_copy(g0, 0).start()

        def unit(g, buf, have_prev_store, prefetch):
            idx_copy(g, buf).wait()
            if prefetch is not None:
                idx_copy(g + 1, 1 - buf).start()
            else:
                @pl.when(g + 1 < g0 + per_w)
                def _():
                    idx_copy(g + 1, 1 - buf).start()
            for fg in range(FG):
                if fg >= 2:
                    out_copy(g, fg).wait()
                else:
                    @pl.when(have_prev_store)
                    def _():
                        out_copy(g, fg).wait()

                @plsc.parallel_loop(0, QB)
                def bt_body(bl):
                    ivs = [idx_v[buf, bl, pl.ds(j * 16, 16)]
                           for j in range(FI)]
                    lookahead = 8
                    n = FI * FI

                    def gath(k):
                        j, fi = divmod(k, FI)
                        return plsc.load_gather(
                            ftab_v.at[pl.ds((fg * FI + fi) * VOCAB, VOCAB)],
                            [ivs[j]])

                    def store(k, val):
                        j, fi = divmod(k, FI)
                        out_v[fg % 2, bl, fi, pl.ds(j * 16, 16)] = val

                    pend = [gath(k) for k in range(lookahead)]
                    for k in range(lookahead, n):
                        pend.append(gath(k))
                        store(k - lookahead, pend[k - lookahead])
                    for k in range(n - lookahead, n):
                        store(k, pend[k])

                out_copy(g, fg).start()

        def body(k, carry):
            g = g0 + k * 2
            unit(g, 0, k > 0, True)
            unit(g + 1, 1, jnp.bool_(True), None)
            return carry

        lax.fori_loop(0, per_w // 2, body, 0)
        g_last = g0 + per_w - 1
        if per_w % 2:
            unit(g_last, 0, jnp.bool_(True), None)
        for fg in (FG - 2, FG - 1):
            out_copy(g_last, fg).wait()

    return gather_k


def kernel(x, table, W, b):
    ftab = _fused_table(table, W, b).reshape(VOCAB * EMBED)
    xl = (x.reshape(BT, BI, TG, TI).transpose(2, 0, 3, 1)
          .astype(jnp.int32))
    out5 = _make_gather()(ftab, xl)
    return out5.transpose(2, 4, 0, 1, 3).reshape(BATCH, TIME, EMBED)
